# Initial kernel scaffold; baseline (speedup 1.0000x reference)
import jax
import jax.numpy as jnp
from jax.experimental import pallas as pl


def kernel(x, embeddings):
    raise NotImplementedError("write your pallas kernel here")


# speedup vs baseline: 2.8676x; 2.8676x over previous
import functools

import jax
import jax.numpy as jnp
from jax.experimental import pallas as pl

SOM0 = 32
SOM1 = 32
CODE_DIM = 32
NCODES = SOM0 * SOM1
BLOCK_B = 256

_HIGH = jax.lax.Precision.HIGHEST


def _tc_body(z_ref, embt_ref, emb_ref,
             zdist_ref, k_ref, zq_ref, zqst_ref,
             up_ref, down_ref, right_ref, left_ref,
             csum_ref, ssum_ref):
    z = z_ref[...]
    embt = embt_ref[...]
    emb = emb_ref[...]

    en = jnp.sum(embt * embt, axis=0, keepdims=True)
    dot = jnp.dot(z, embt, preferred_element_type=jnp.float32,
                  precision=_HIGH)
    s = en - 2.0 * dot
    zn = jnp.sum(z * z, axis=1, keepdims=True)
    zdist_ref[...] = zn + s

    idx = jax.lax.broadcasted_iota(jnp.int32, s.shape, 1)
    m1 = jnp.min(s, axis=1, keepdims=True)
    j1 = jnp.min(jnp.where(s == m1, idx, NCODES), axis=1, keepdims=True)
    s2 = jnp.where(idx == j1, jnp.float32(3e38), s)
    m2 = jnp.min(s2, axis=1, keepdims=True)
    j2 = jnp.min(jnp.where(s2 == m2, idx, NCODES), axis=1, keepdims=True)

    oh1 = (idx == j1).astype(jnp.float32)
    oh2 = (idx == j2).astype(jnp.float32)
    e1 = jnp.dot(oh1, emb, preferred_element_type=jnp.float32, precision=_HIGH)
    e2 = jnp.dot(oh2, emb, preferred_element_type=jnp.float32, precision=_HIGH)
    d1 = jnp.sum((z - e1) ** 2, axis=1, keepdims=True)
    d2 = jnp.sum((z - e2) ** 2, axis=1, keepdims=True)
    take2 = (d2 < d1) | ((d2 == d1) & (j2 < j1))
    k = jnp.where(take2, j2, j1)
    k_ref[...] = k
    zq = jnp.where(take2, e2, e1)
    zq_ref[...] = zq
    zqst_ref[...] = z + (zq - z)

    k1 = k >> 5
    k2 = k & 31
    snb = jnp.float32(0.0)
    for out_ref, j_n, mask_b in (
            (up_ref, jnp.where(k1 < SOM0 - 1, k + SOM1, k), k1 < SOM0 - 1),
            (down_ref, jnp.where(k1 > 0, k - SOM1, k), k1 > 0),
            (right_ref, jnp.where(k2 < SOM1 - 1, k + 1, k), k2 < SOM1 - 1),
            (left_ref, jnp.where(k2 > 0, k - 1, k), k2 > 0)):
        oh = (idx == j_n).astype(jnp.float32)
        row = jnp.dot(oh, emb, preferred_element_type=jnp.float32,
                      precision=_HIGH)
        row = row * mask_b.astype(jnp.float32)
        out_ref[...] = row
        snb = snb + jnp.sum((z - row) ** 2)

    part = jnp.sum((zq - z) ** 2)

    @pl.when(pl.program_id(0) == 0)
    def _init():
        csum_ref[...] = jnp.zeros((1, 1), jnp.float32)
        ssum_ref[...] = jnp.zeros((1, 1), jnp.float32)

    csum_ref[...] += part.reshape(1, 1)
    ssum_ref[...] += snb.reshape(1, 1)


@functools.partial(jax.jit, static_argnames=())
def kernel(x, embeddings):
    n, c, t = x.shape
    b = n * t
    z_e = jnp.transpose(x, (0, 2, 1)).reshape(b, c)
    emb = embeddings.reshape(NCODES, CODE_DIM)
    embt = emb.T

    grid = (b // BLOCK_B,)
    out_shapes = (
        jax.ShapeDtypeStruct((b, NCODES), jnp.float32),
        jax.ShapeDtypeStruct((b, 1), jnp.int32),
        jax.ShapeDtypeStruct((b, CODE_DIM), jnp.float32),
        jax.ShapeDtypeStruct((b, CODE_DIM), jnp.float32),
        jax.ShapeDtypeStruct((b, CODE_DIM), jnp.float32),
        jax.ShapeDtypeStruct((b, CODE_DIM), jnp.float32),
        jax.ShapeDtypeStruct((b, CODE_DIM), jnp.float32),
        jax.ShapeDtypeStruct((b, CODE_DIM), jnp.float32),
        jax.ShapeDtypeStruct((1, 1), jnp.float32),
        jax.ShapeDtypeStruct((1, 1), jnp.float32),
    )
    row_spec = pl.BlockSpec((BLOCK_B, CODE_DIM), lambda i: (i, 0))
    out_specs = (
        pl.BlockSpec((BLOCK_B, NCODES), lambda i: (i, 0)),
        pl.BlockSpec((BLOCK_B, 1), lambda i: (i, 0)),
        row_spec, row_spec, row_spec, row_spec, row_spec, row_spec,
        pl.BlockSpec((1, 1), lambda i: (0, 0)),
        pl.BlockSpec((1, 1), lambda i: (0, 0)),
    )
    in_specs = (
        row_spec,
        pl.BlockSpec((CODE_DIM, NCODES), lambda i: (0, 0)),
        pl.BlockSpec((NCODES, CODE_DIM), lambda i: (0, 0)),
    )
    (z_dist, k2d, z_q, z_q_st, up, down, right, left,
     csum, ssum) = pl.pallas_call(
        _tc_body,
        grid=grid,
        in_specs=in_specs,
        out_specs=out_specs,
        out_shape=out_shapes,
    )(z_e, embt, emb)

    k = k2d.reshape(b)
    z_q_neighbors = jnp.stack([z_q, up, down, right, left], axis=1)
    commit_l = 2.0 * (csum[0, 0] / jnp.float32(b * c))
    som_l = (csum[0, 0] + ssum[0, 0]) / jnp.float32(b * 5 * c)
    z_q_out = jnp.transpose(z_q_st.reshape(n, t, c), (0, 2, 1))
    return (z_q_out, commit_l, som_l, z_q_neighbors, z_dist, k)


# speedup vs baseline: 4.4679x; 1.5581x over previous
import functools

import jax
import jax.numpy as jnp
from jax.experimental import pallas as pl
from jax.experimental.pallas import tpu as pltpu

SOM0 = 32
SOM1 = 32
CODE_DIM = 32
NCODES = SOM0 * SOM1
BLOCK_B = 256

_HIGH = jax.lax.Precision.HIGHEST

_NTAB = 11


def _tc_body(z_ref, embt_ref, tab_ref,
             zdist_ref, k_ref, zq_ref, zqst_ref,
             up_ref, down_ref, right_ref, left_ref,
             csum_ref, ssum_ref, en_ref):
    z = z_ref[...]

    @pl.when(pl.program_id(0) == 0)
    def _compute_en():
        embt = embt_ref[...]
        en_ref[...] = jnp.sum(embt * embt, axis=0, keepdims=True)

    en = en_ref[...]
    dot = jnp.dot(z, embt_ref[...], preferred_element_type=jnp.float32,
                  precision=_HIGH)
    s = en - 2.0 * dot
    zn = jnp.sum(z * z, axis=1, keepdims=True)
    zdist_ref[...] = zn + s

    idx = jax.lax.broadcasted_iota(jnp.int32, s.shape, 1)
    m1 = jnp.min(s, axis=1, keepdims=True)
    j1 = jnp.min(jnp.where(s == m1, idx, NCODES), axis=1, keepdims=True)
    s2 = jnp.where(idx == j1, jnp.float32(3e38), s)
    m2 = jnp.min(s2, axis=1, keepdims=True)
    j2 = jnp.min(jnp.where(s2 == m2, idx, NCODES), axis=1, keepdims=True)

    oh1 = (idx == j1).astype(jnp.bfloat16)
    oh2 = (idx == j2).astype(jnp.bfloat16)
    oh = jnp.concatenate([oh1, oh2], axis=0)
    g = jnp.dot(oh, tab_ref[...], preferred_element_type=jnp.float32)
    bb = z.shape[0]
    g1 = g[:bb]
    g2 = g[bb:]

    def _sect(gg, i):
        return gg[:, i * CODE_DIM:(i + 1) * CODE_DIM]

    e1 = (_sect(g1, 0) + _sect(g1, 1)) + _sect(g1, 2)
    e2 = (_sect(g2, 0) + _sect(g2, 1)) + _sect(g2, 2)
    d1 = jnp.sum((z - e1) ** 2, axis=1, keepdims=True)
    d2 = jnp.sum((z - e2) ** 2, axis=1, keepdims=True)
    take2 = (d2 < d1) | ((d2 == d1) & (j2 < j1))
    k = jnp.where(take2, j2, j1)
    k_ref[...] = k
    zq = jnp.where(take2, e2, e1)
    zq_ref[...] = zq
    zqst_ref[...] = z + (zq - z)

    k1 = k >> 5
    k2 = k & 31
    snb = jnp.float32(0.0)
    for out_ref, sect_i, mask_b in (
            (up_ref, 3, k1 < SOM0 - 1),
            (down_ref, 5, k1 > 0),
            (right_ref, 7, k2 < SOM1 - 1),
            (left_ref, 9, k2 > 0)):
        row1 = _sect(g1, sect_i) + _sect(g1, sect_i + 1)
        row2 = _sect(g2, sect_i) + _sect(g2, sect_i + 1)
        row = jnp.where(take2, row2, row1)
        row = row * mask_b.astype(jnp.float32)
        out_ref[...] = row
        snb = snb + jnp.sum((z - row) ** 2)

    part = jnp.sum((zq - z) ** 2)

    @pl.when(pl.program_id(0) == 0)
    def _init():
        csum_ref[...] = jnp.zeros((1, 1), jnp.float32)
        ssum_ref[...] = jnp.zeros((1, 1), jnp.float32)

    csum_ref[...] += part.reshape(1, 1)
    ssum_ref[...] += snb.reshape(1, 1)


@functools.partial(jax.jit, static_argnames=())
def kernel(x, embeddings):
    n, c, t = x.shape
    b = n * t
    z_e = jnp.transpose(x, (0, 2, 1)).reshape(b, c)
    emb = embeddings.reshape(NCODES, CODE_DIM)
    embt = emb.T

    hi16 = emb.astype(jnp.bfloat16)
    hi = hi16.astype(jnp.float32)
    mid16 = (emb - hi).astype(jnp.bfloat16)
    mid = mid16.astype(jnp.float32)
    lo16 = (emb - hi - mid).astype(jnp.bfloat16)

    def _roll2(shift):
        return (jnp.roll(hi16, shift, axis=0), jnp.roll(mid16, shift, axis=0))

    up_h, up_m = _roll2(-SOM1)
    dn_h, dn_m = _roll2(SOM1)
    rt_h, rt_m = _roll2(-1)
    lf_h, lf_m = _roll2(1)
    tab = jnp.concatenate(
        [hi16, mid16, lo16, up_h, up_m, dn_h, dn_m, rt_h, rt_m, lf_h, lf_m],
        axis=1)

    grid = (b // BLOCK_B,)
    out_shapes = (
        jax.ShapeDtypeStruct((b, NCODES), jnp.float32),
        jax.ShapeDtypeStruct((b, 1), jnp.int32),
        jax.ShapeDtypeStruct((b, CODE_DIM), jnp.float32),
        jax.ShapeDtypeStruct((b, CODE_DIM), jnp.float32),
        jax.ShapeDtypeStruct((b, CODE_DIM), jnp.float32),
        jax.ShapeDtypeStruct((b, CODE_DIM), jnp.float32),
        jax.ShapeDtypeStruct((b, CODE_DIM), jnp.float32),
        jax.ShapeDtypeStruct((b, CODE_DIM), jnp.float32),
        jax.ShapeDtypeStruct((1, 1), jnp.float32),
        jax.ShapeDtypeStruct((1, 1), jnp.float32),
    )
    row_spec = pl.BlockSpec((BLOCK_B, CODE_DIM), lambda i: (i, 0))
    out_specs = (
        pl.BlockSpec((BLOCK_B, NCODES), lambda i: (i, 0)),
        pl.BlockSpec((BLOCK_B, 1), lambda i: (i, 0)),
        row_spec, row_spec, row_spec, row_spec, row_spec, row_spec,
        pl.BlockSpec((1, 1), lambda i: (0, 0)),
        pl.BlockSpec((1, 1), lambda i: (0, 0)),
    )
    in_specs = (
        row_spec,
        pl.BlockSpec((CODE_DIM, NCODES), lambda i: (0, 0)),
        pl.BlockSpec((NCODES, _NTAB * CODE_DIM), lambda i: (0, 0)),
    )
    (z_dist, k2d, z_q, z_q_st, up, down, right, left,
     csum, ssum) = pl.pallas_call(
        _tc_body,
        grid=grid,
        in_specs=in_specs,
        out_specs=out_specs,
        out_shape=out_shapes,
        scratch_shapes=[pltpu.VMEM((1, NCODES), jnp.float32)],
    )(z_e, embt, tab)

    k = k2d.reshape(b)
    z_q_neighbors = jnp.stack([z_q, up, down, right, left], axis=1)
    commit_l = 2.0 * (csum[0, 0] / jnp.float32(b * c))
    som_l = (csum[0, 0] + ssum[0, 0]) / jnp.float32(b * 5 * c)
    z_q_out = jnp.transpose(z_q_st.reshape(n, t, c), (0, 2, 1))
    return (z_q_out, commit_l, som_l, z_q_neighbors, z_dist, k)
